# Initial kernel scaffold; baseline (speedup 1.0000x reference)
import jax
import jax.numpy as jnp
from jax.experimental import pallas as pl


def kernel(input_embeddings, parent, root, W_iou, b_iou, U_iou, W_f, b_f, U_f):
    raise NotImplementedError("write your pallas kernel here")


# speedup vs baseline: 71.8082x; 71.8082x over previous
import jax
import jax.numpy as jnp
from jax.experimental import pallas as pl
from jax.experimental.pallas import tpu as pltpu

N_NODES = 10000
D_IN = 128
H_DIM = 128
NPAD = 10240


def _proj_body(x_ref, w_ref, b_ref, oi_ref, oo_ref, ou_ref, of_ref):
    p = (jnp.dot(x_ref[...], w_ref[...], preferred_element_type=jnp.float32)
         + b_ref[...])
    oi_ref[...] = p[:, 0:128]
    oo_ref[...] = p[:, 128:256]
    ou_ref[...] = p[:, 256:384]
    of_ref[...] = p[:, 384:512]


def _tree_body(xig_ref, xog_ref, xug_ref, xf_ref, uiou_ref, uf_ref,
               order_ref, spar_ref, lvl_ref, prm_ref, h_out, c_out,
               hsum, fcsum, a_s, xig_s, xog_s, xug_s, xf_s, fc_s):
    n_levels = prm_ref[0]

    def _zero(i, _):
        z = jnp.zeros((1024, H_DIM), dtype=jnp.float32)
        hsum[pl.ds(i * 1024, 1024), :] = z
        fcsum[pl.ds(i * 1024, 1024), :] = z
        return 0

    jax.lax.fori_loop(0, NPAD // 1024, _zero, 0)

    def _level(t, _):
        s = lvl_ref[t]
        e = lvl_ref[t + 1]
        nch = (e - s + 7) // 8

        def _chunk(ci, carry):
            base = s + ci * 8
            for r in range(8):
                gpos = base + r

                @pl.when(gpos < e)
                def _gather():
                    nd = order_ref[gpos]
                    p = spar_ref[gpos]
                    a_s[r:r + 1, :] = hsum[pl.ds(nd, 1), :]
                    fc_s[r:r + 1, :] = fcsum[pl.ds(nd, 1), :]
                    xig_s[r:r + 1, :] = xig_ref[pl.ds(nd, 1), :]
                    xog_s[r:r + 1, :] = xog_ref[pl.ds(nd, 1), :]
                    xug_s[r:r + 1, :] = xug_ref[pl.ds(nd, 1), :]
                    xf_s[r:r + 1, :] = xf_ref[pl.ds(p, 1), :]

            iou = jnp.dot(a_s[...], uiou_ref[...],
                          preferred_element_type=jnp.float32)
            ig = xig_s[...] + iou[:, 0:128]
            og = xog_s[...] + iou[:, 128:256]
            ug = xug_s[...] + iou[:, 256:384]
            c = jax.nn.sigmoid(ig) * jnp.tanh(ug) + fc_s[...]
            h = jax.nn.sigmoid(og) * jnp.tanh(c)
            uf = jnp.dot(h, uf_ref[...], preferred_element_type=jnp.float32)
            f = jax.nn.sigmoid(xf_s[...] + uf)
            fcc = f * c

            for r in range(8):
                gpos = base + r

                @pl.when(gpos < e)
                def _scatter():
                    nd = order_ref[gpos]
                    p = spar_ref[gpos]
                    h_out[pl.ds(nd, 1), :] = h[r:r + 1, :]
                    c_out[pl.ds(nd, 1), :] = c[r:r + 1, :]
                    hsum[pl.ds(p, 1), :] = hsum[pl.ds(p, 1), :] + h[r:r + 1, :]
                    fcsum[pl.ds(p, 1), :] = (
                        fcsum[pl.ds(p, 1), :] + fcc[r:r + 1, :])

            return carry

        jax.lax.fori_loop(0, nch, _chunk, 0)
        return 0

    jax.lax.fori_loop(0, n_levels, _level, 0)


def kernel(input_embeddings, parent, root, W_iou, b_iou, U_iou, W_f, b_f, U_f):
    del root
    x = input_embeddings.astype(jnp.float32)
    n = N_NODES
    idx = jnp.arange(n, dtype=jnp.int32)
    par = parent.astype(jnp.int32)

    d = (idx > 0).astype(jnp.int32)
    ptr = par
    for _ in range(14):
        d = d + d[ptr]
        ptr = ptr[ptr]
    maxd = jnp.max(d)
    order = jnp.argsort(-d).astype(jnp.int32)
    cnt = jnp.zeros((n,), jnp.int32).at[d].add(1)
    cnt_desc = jnp.where(idx <= maxd, cnt[jnp.clip(maxd - idx, 0, n - 1)], 0)
    lvl = jnp.concatenate([
        jnp.zeros((1,), jnp.int32),
        jnp.cumsum(cnt_desc).astype(jnp.int32),
        jnp.full((15,), n, jnp.int32),
    ])
    spar = jnp.where(order == 0, N_NODES, par[order]).astype(jnp.int32)
    prm = (maxd + 1).reshape(1)

    wcat = jnp.concatenate([W_iou.T, W_f.T], axis=1).astype(jnp.float32)
    bcat = jnp.concatenate([b_iou, b_f]).reshape(1, 4 * H_DIM)
    xpad = jnp.zeros((NPAD, D_IN), jnp.float32).at[:n].set(x)
    xig, xog, xug, xf = pl.pallas_call(
        _proj_body,
        grid=(NPAD // 1024,),
        in_specs=[
            pl.BlockSpec((1024, D_IN), lambda i: (i, 0)),
            pl.BlockSpec((D_IN, 4 * H_DIM), lambda i: (0, 0)),
            pl.BlockSpec((1, 4 * H_DIM), lambda i: (0, 0)),
        ],
        out_specs=tuple(
            pl.BlockSpec((1024, H_DIM), lambda i: (i, 0)) for _ in range(4)),
        out_shape=tuple(
            jax.ShapeDtypeStruct((NPAD, H_DIM), jnp.float32)
            for _ in range(4)),
    )(xpad, wcat, bcat)

    h, c = pl.pallas_call(
        _tree_body,
        in_specs=(
            [pl.BlockSpec(memory_space=pltpu.VMEM) for _ in range(6)]
            + [pl.BlockSpec(memory_space=pltpu.SMEM) for _ in range(4)]
        ),
        out_specs=(
            pl.BlockSpec(memory_space=pltpu.VMEM),
            pl.BlockSpec(memory_space=pltpu.VMEM),
        ),
        out_shape=(
            jax.ShapeDtypeStruct((N_NODES, H_DIM), jnp.float32),
            jax.ShapeDtypeStruct((N_NODES, H_DIM), jnp.float32),
        ),
        scratch_shapes=(
            [pltpu.VMEM((NPAD, H_DIM), jnp.float32) for _ in range(2)]
            + [pltpu.VMEM((8, H_DIM), jnp.float32) for _ in range(6)]
        ),
        compiler_params=pltpu.CompilerParams(
            vmem_limit_bytes=100 * 1024 * 1024,
        ),
    )(xig, xog, xug, xf, U_iou.T.astype(jnp.float32),
      U_f.T.astype(jnp.float32), order, spar, lvl, prm)
    return h, c


# speedup vs baseline: 75.6048x; 1.0529x over previous
import jax
import jax.numpy as jnp
from jax.experimental import pallas as pl
from jax.experimental.pallas import tpu as pltpu

N_NODES = 10000
D_IN = 128
H_DIM = 128
NPAD = 10240


def _proj_body(x_ref, w_ref, b_ref, oiou_ref, of_ref):
    p = (jnp.dot(x_ref[...], w_ref[...], preferred_element_type=jnp.float32)
         + b_ref[...])
    oiou_ref[...] = p[:, 0:384]
    of_ref[...] = p[:, 384:512]


def _tree_body(xiou_ref, xf_ref, uiou_ref, uf_ref,
               order_ref, spar_ref, lvl_ref, prm_ref, h_out, c_out,
               hfc, hc_s, st_s, xi_s, xp_s):
    n_levels = prm_ref[0]

    def _zero(i, _):
        hfc[pl.ds(i * 1024, 1024), :] = jnp.zeros((1024, 2 * H_DIM),
                                                  dtype=jnp.float32)
        return 0

    jax.lax.fori_loop(0, NPAD // 1024, _zero, 0, unroll=True)

    def _level(t, _):
        s = lvl_ref[t]
        e = lvl_ref[t + 1]
        last = e - 1
        nch = (e - s + 7) // 8

        def _chunk(ci, carry):
            base = s + ci * 8
            nds = []
            ps = []
            for r in range(8):
                gpos = base + r
                safe = jnp.minimum(gpos, last)
                nd = order_ref[safe]
                p = jnp.where(gpos < e, spar_ref[safe], N_NODES)
                nds.append(nd)
                ps.append(p)
                st_s[r:r + 1, :] = hfc[pl.ds(nd, 1), :]
                xi_s[r:r + 1, :] = xiou_ref[pl.ds(nd, 1), :]
                xp_s[r:r + 1, :] = xf_ref[pl.ds(p, 1), :]

            st = st_s[...]
            iou = xi_s[...] + jnp.dot(st[:, 0:128], uiou_ref[...],
                                      preferred_element_type=jnp.float32)
            c = (jax.nn.sigmoid(iou[:, 0:128]) * jnp.tanh(iou[:, 256:384])
                 + st[:, 128:256])
            h = jax.nn.sigmoid(iou[:, 128:256]) * jnp.tanh(c)
            uf = jnp.dot(h, uf_ref[...], preferred_element_type=jnp.float32)
            f = jax.nn.sigmoid(xp_s[...] + uf)
            hc = jnp.concatenate([h, c], axis=1)
            contrib = jnp.concatenate([h, f * c], axis=1)

            for r in range(8):
                hc_s[pl.ds(nds[r], 1), :] = hc[r:r + 1, :]
                hfc[pl.ds(ps[r], 1), :] = (
                    hfc[pl.ds(ps[r], 1), :] + contrib[r:r + 1, :])
            return carry

        jax.lax.fori_loop(0, nch, _chunk, 0)
        return 0

    jax.lax.fori_loop(0, n_levels, _level, 0)

    rows = 1000 if N_NODES % 1000 == 0 else N_NODES

    def _split(i, _):
        blk = hc_s[pl.ds(i * rows, rows), :]
        h_out[pl.ds(i * rows, rows), :] = blk[:, 0:128]
        c_out[pl.ds(i * rows, rows), :] = blk[:, 128:256]
        return 0

    jax.lax.fori_loop(0, N_NODES // rows, _split, 0, unroll=True)


def kernel(input_embeddings, parent, root, W_iou, b_iou, U_iou, W_f, b_f, U_f):
    del root
    x = input_embeddings.astype(jnp.float32)
    n = N_NODES
    idx = jnp.arange(n, dtype=jnp.int32)
    par = parent.astype(jnp.int32)

    d = (idx > 0).astype(jnp.int32)
    ptr = par
    for _ in range(14):
        d = d + d[ptr]
        ptr = ptr[ptr]
    maxd = jnp.max(d)
    order = jnp.argsort(-d).astype(jnp.int32)
    cnt = jnp.zeros((n,), jnp.int32).at[d].add(1)
    cnt_desc = jnp.where(idx <= maxd, cnt[jnp.clip(maxd - idx, 0, n - 1)], 0)
    lvl = jnp.concatenate([
        jnp.zeros((1,), jnp.int32),
        jnp.cumsum(cnt_desc).astype(jnp.int32),
        jnp.full((15,), n, jnp.int32),
    ])
    spar = jnp.where(order == 0, N_NODES, par[order]).astype(jnp.int32)
    prm = (maxd + 1).reshape(1)

    wcat = jnp.concatenate([W_iou.T, W_f.T], axis=1).astype(jnp.float32)
    bcat = jnp.concatenate([b_iou, b_f]).reshape(1, 4 * H_DIM)
    xpad = jnp.zeros((NPAD, D_IN), jnp.float32).at[:n].set(x)
    xiou, xf = pl.pallas_call(
        _proj_body,
        grid=(NPAD // 1024,),
        in_specs=[
            pl.BlockSpec((1024, D_IN), lambda i: (i, 0)),
            pl.BlockSpec((D_IN, 4 * H_DIM), lambda i: (0, 0)),
            pl.BlockSpec((1, 4 * H_DIM), lambda i: (0, 0)),
        ],
        out_specs=(
            pl.BlockSpec((1024, 3 * H_DIM), lambda i: (i, 0)),
            pl.BlockSpec((1024, H_DIM), lambda i: (i, 0)),
        ),
        out_shape=(
            jax.ShapeDtypeStruct((NPAD, 3 * H_DIM), jnp.float32),
            jax.ShapeDtypeStruct((NPAD, H_DIM), jnp.float32),
        ),
    )(xpad, wcat, bcat)

    h, c = pl.pallas_call(
        _tree_body,
        in_specs=(
            [pl.BlockSpec(memory_space=pltpu.VMEM) for _ in range(4)]
            + [pl.BlockSpec(memory_space=pltpu.SMEM) for _ in range(4)]
        ),
        out_specs=(
            pl.BlockSpec(memory_space=pltpu.VMEM),
            pl.BlockSpec(memory_space=pltpu.VMEM),
        ),
        out_shape=(
            jax.ShapeDtypeStruct((N_NODES, H_DIM), jnp.float32),
            jax.ShapeDtypeStruct((N_NODES, H_DIM), jnp.float32),
        ),
        scratch_shapes=[
            pltpu.VMEM((NPAD, 2 * H_DIM), jnp.float32),
            pltpu.VMEM((NPAD, 2 * H_DIM), jnp.float32),
            pltpu.VMEM((8, 2 * H_DIM), jnp.float32),
            pltpu.VMEM((8, 3 * H_DIM), jnp.float32),
            pltpu.VMEM((8, H_DIM), jnp.float32),
        ],
        compiler_params=pltpu.CompilerParams(
            vmem_limit_bytes=100 * 1024 * 1024,
        ),
    )(xiou, xf, U_iou.T.astype(jnp.float32), U_f.T.astype(jnp.float32),
      order, spar, lvl, prm)
    return h, c


# speedup vs baseline: 225.3121x; 2.9801x over previous
import jax
import jax.numpy as jnp
from jax import lax
from jax.experimental import pallas as pl
from jax.experimental.pallas import tpu as pltpu
from jax.experimental.pallas import tpu_sc as plsc

N_NODES = 10000
D_IN = 128
H_DIM = 128
NPAD = 10240
NQPAD = 20480


def _depth_body(par_hbm, d_out, pa, pb, da, db):
    cid = lax.axis_index("c")
    sid = lax.axis_index("s")
    wid = sid * 2 + cid
    pltpu.sync_copy(par_hbm, pa)

    def _init(i, _):
        da[pl.ds(i * 16, 16)] = jnp.ones((16,), jnp.int32)
        return 0

    lax.fori_loop(0, NPAD // 16, _init, 0)
    da[pl.ds(0, 16)] = jnp.minimum(lax.iota(jnp.int32, 16), 1)

    for k in range(14):
        sd, sp, dd, dp = ((da, pa, db, pb) if k % 2 == 0
                          else (db, pb, da, pa))

        def _round(i, _, sd=sd, sp=sp, dd=dd, dp=dp):
            sl = pl.ds(i * 16, 16)
            idxv = sp[sl]
            dd[sl] = sd[sl] + plsc.load_gather(sd, [idxv])
            dp[sl] = plsc.load_gather(sp, [idxv])
            return 0

        lax.fori_loop(0, NPAD // 16, _round, 0)
    sl_out = pl.ds(wid * (NPAD // 32), NPAD // 32)
    pltpu.sync_copy(da.at[sl_out], d_out.at[sl_out])


def _depth_sc(par_pad):
    mesh = plsc.VectorSubcoreMesh(core_axis_name="c", subcore_axis_name="s")
    fn = pl.kernel(
        _depth_body,
        mesh=mesh,
        out_type=jax.ShapeDtypeStruct((NPAD,), jnp.int32),
        scratch_types=[pltpu.VMEM((NPAD,), jnp.int32) for _ in range(4)],
        compiler_params=pltpu.CompilerParams(needs_layout_passes=False),
    )
    return fn(par_pad)


def _search_body(keys_hbm, q_hbm, out_hbm, kv, qv, ov):
    cid = lax.axis_index("c")
    sid = lax.axis_index("s")
    wid = sid * 2 + cid
    per_w = NQPAD // 32
    base = wid * per_w
    pltpu.sync_copy(keys_hbm, kv)
    pltpu.sync_copy(q_hbm.at[pl.ds(base, per_w)], qv)

    def _srch(i, _):
        sl = pl.ds(i * 16, 16)
        q = qv[sl]
        pos = jnp.zeros((16,), jnp.int32)
        for step in (8192, 4096, 2048, 1024, 512, 256, 128, 64,
                     32, 16, 8, 4, 2, 1):
            npos = pos + step
            ok = npos <= N_NODES
            idx = jnp.minimum(npos, N_NODES) - 1
            kvals = plsc.load_gather(kv, [idx])
            pos = jnp.where(ok & (kvals < q), npos, pos)
        ov[sl] = pos
        return 0

    lax.fori_loop(0, per_w // 16, _srch, 0)
    pltpu.sync_copy(ov, out_hbm.at[pl.ds(base, per_w)])


def _search_sc(keys_sorted, queries):
    mesh = plsc.VectorSubcoreMesh(core_axis_name="c", subcore_axis_name="s")
    fn = pl.kernel(
        _search_body,
        mesh=mesh,
        out_type=jax.ShapeDtypeStruct((NQPAD,), jnp.int32),
        scratch_types=[
            pltpu.VMEM((N_NODES,), jnp.int32),
            pltpu.VMEM((NQPAD // 32,), jnp.int32),
            pltpu.VMEM((NQPAD // 32,), jnp.int32),
        ],
        compiler_params=pltpu.CompilerParams(needs_layout_passes=False),
    )
    return fn(keys_sorted, queries)


def _proj_body(x_ref, w_ref, b_ref, oiou_ref, of_ref):
    p = (jnp.dot(x_ref[...], w_ref[...], preferred_element_type=jnp.float32)
         + b_ref[...])
    oiou_ref[...] = p[:, 0:384]
    of_ref[...] = p[:, 384:512]


def _tree_body(xiou_ref, xf_ref, uiou_ref, uf_ref,
               order_ref, spar_ref, cs_ref, ce_ref, lvl_ref, prm_ref,
               h_out, c_out, pref, hc_s, st_s, xi_s, xp_s):
    n_levels = prm_ref[0]

    pref[0:8, :] = jnp.zeros((8, 2 * H_DIM), dtype=jnp.float32)

    def _level(t, carry):
        s = lvl_ref[t]
        e = lvl_ref[t + 1]
        last = e - 1
        nch = (e - s + 7) // 8

        def _chunk(ci, cin):
            base = s + ci * 8
            nds = []
            for r in range(8):
                gpos = base + r
                safe = jnp.minimum(gpos, last)
                nd = order_ref[safe]
                p = spar_ref[safe]
                a0 = cs_ref[safe]
                a1 = ce_ref[safe]
                nds.append(nd)
                st_s[r:r + 1, :] = (pref[pl.ds(a1, 1), :]
                                    - pref[pl.ds(a0, 1), :])
                xi_s[r:r + 1, :] = xiou_ref[pl.ds(nd, 1), :]
                xp_s[r:r + 1, :] = xf_ref[pl.ds(p, 1), :]

            st = st_s[...]
            iou = xi_s[...] + jnp.dot(st[:, 0:128], uiou_ref[...],
                                      preferred_element_type=jnp.float32)
            c = (jax.nn.sigmoid(iou[:, 0:128]) * jnp.tanh(iou[:, 256:384])
                 + st[:, 128:256])
            h = jax.nn.sigmoid(iou[:, 128:256]) * jnp.tanh(c)
            uf = jnp.dot(h, uf_ref[...], preferred_element_type=jnp.float32)
            f = jax.nn.sigmoid(xp_s[...] + uf)
            hc = jnp.concatenate([h, c], axis=1)
            contrib = jnp.concatenate([h, f * c], axis=1)

            rows = jax.lax.broadcasted_iota(jnp.int32, (8, 1), 0)
            contrib = jnp.where(rows < (e - base), contrib, 0.0)

            z = jnp.zeros_like(contrib)
            x1 = contrib
            x2 = x1 + jnp.concatenate([z[0:1], x1[0:7]], axis=0)
            x3 = x2 + jnp.concatenate([z[0:2], x2[0:6]], axis=0)
            cum = x3 + jnp.concatenate([z[0:4], x3[0:4]], axis=0) + cin

            for r in range(8):
                pref[pl.ds(base + r + 1, 1), :] = cum[r:r + 1, :]
                hc_s[pl.ds(nds[r], 1), :] = hc[r:r + 1, :]
            return cum[7:8, :]

        return jax.lax.fori_loop(0, nch, _chunk, carry)

    jax.lax.fori_loop(0, n_levels, _level,
                      jnp.zeros((1, 2 * H_DIM), jnp.float32))

    rows = 1000 if N_NODES % 1000 == 0 else N_NODES

    def _split(i, _):
        blk = hc_s[pl.ds(i * rows, rows), :]
        h_out[pl.ds(i * rows, rows), :] = blk[:, 0:128]
        c_out[pl.ds(i * rows, rows), :] = blk[:, 128:256]
        return 0

    jax.lax.fori_loop(0, N_NODES // rows, _split, 0, unroll=True)


def kernel(input_embeddings, parent, root, W_iou, b_iou, U_iou, W_f, b_f, U_f):
    del root
    x = input_embeddings.astype(jnp.float32)
    n = N_NODES
    idx = jnp.arange(n, dtype=jnp.int32)
    par = parent.astype(jnp.int32)

    d = _depth_sc(jnp.zeros((NPAD,), jnp.int32).at[:n].set(par))[:n]
    maxd = jnp.max(d)
    childcnt = jnp.zeros((n,), jnp.int32).at[par[1:]].add(1)
    key = (maxd - d) * n + par
    keys_s, order, spar_raw, ccnt_s, d_s = lax.sort(
        (key, idx, par, childcnt, d), num_keys=1)
    spar = jnp.where(order == 0, N_NODES, spar_raw)
    qk_cs = (maxd - d_s - 1) * n + order
    qk_lvl = jnp.arange(n + 16, dtype=jnp.int32) * n
    queries = jnp.concatenate([
        qk_cs, qk_lvl,
        jnp.zeros((NQPAD - 2 * n - 16,), jnp.int32),
    ])
    res = _search_sc(keys_s, queries)
    cs_s = res[:n]
    ce_s = cs_s + ccnt_s
    lvl = res[n:2 * n + 16]
    prm = (maxd + 1).reshape(1)

    wcat = jnp.concatenate([W_iou.T, W_f.T], axis=1).astype(jnp.float32)
    bcat = jnp.concatenate([b_iou, b_f]).reshape(1, 4 * H_DIM)
    xpad = jnp.zeros((NPAD, D_IN), jnp.float32).at[:n].set(x)
    xiou, xf = pl.pallas_call(
        _proj_body,
        grid=(NPAD // 1024,),
        in_specs=[
            pl.BlockSpec((1024, D_IN), lambda i: (i, 0)),
            pl.BlockSpec((D_IN, 4 * H_DIM), lambda i: (0, 0)),
            pl.BlockSpec((1, 4 * H_DIM), lambda i: (0, 0)),
        ],
        out_specs=(
            pl.BlockSpec((1024, 3 * H_DIM), lambda i: (i, 0)),
            pl.BlockSpec((1024, H_DIM), lambda i: (i, 0)),
        ),
        out_shape=(
            jax.ShapeDtypeStruct((NPAD, 3 * H_DIM), jnp.float32),
            jax.ShapeDtypeStruct((NPAD, H_DIM), jnp.float32),
        ),
    )(xpad, wcat, bcat)

    h, c = pl.pallas_call(
        _tree_body,
        in_specs=(
            [pl.BlockSpec(memory_space=pltpu.VMEM) for _ in range(4)]
            + [pl.BlockSpec(memory_space=pltpu.SMEM) for _ in range(6)]
        ),
        out_specs=(
            pl.BlockSpec(memory_space=pltpu.VMEM),
            pl.BlockSpec(memory_space=pltpu.VMEM),
        ),
        out_shape=(
            jax.ShapeDtypeStruct((N_NODES, H_DIM), jnp.float32),
            jax.ShapeDtypeStruct((N_NODES, H_DIM), jnp.float32),
        ),
        scratch_shapes=[
            pltpu.VMEM((NPAD, 2 * H_DIM), jnp.float32),
            pltpu.VMEM((NPAD, 2 * H_DIM), jnp.float32),
            pltpu.VMEM((8, 2 * H_DIM), jnp.float32),
            pltpu.VMEM((8, 3 * H_DIM), jnp.float32),
            pltpu.VMEM((8, H_DIM), jnp.float32),
        ],
        compiler_params=pltpu.CompilerParams(
            vmem_limit_bytes=100 * 1024 * 1024,
        ),
    )(xiou, xf, U_iou.T.astype(jnp.float32), U_f.T.astype(jnp.float32),
      order, spar, cs_s, ce_s, lvl, prm)
    return h, c


# speedup vs baseline: 325.0279x; 1.4426x over previous
import jax
import jax.numpy as jnp
from jax import lax
from jax.experimental import pallas as pl
from jax.experimental.pallas import tpu as pltpu
from jax.experimental.pallas import tpu_sc as plsc

N_NODES = 10000
D_IN = 128
H_DIM = 128
NPAD = 10240
NQPAD = 30720
CHUNK = 16


def _depth_body(par_hbm, d_out, pa, pb, da, db):
    cid = lax.axis_index("c")
    sid = lax.axis_index("s")
    wid = sid * 2 + cid
    pltpu.sync_copy(par_hbm, pa)

    def _init(i, _):
        da[pl.ds(i * 16, 16)] = jnp.ones((16,), jnp.int32)
        return 0

    lax.fori_loop(0, NPAD // 16, _init, 0)
    da[pl.ds(0, 16)] = jnp.minimum(lax.iota(jnp.int32, 16), 1)

    for k in range(14):
        sd, sp, dd, dp = ((da, pa, db, pb) if k % 2 == 0
                          else (db, pb, da, pa))

        def _round(i, _, sd=sd, sp=sp, dd=dd, dp=dp):
            sl = pl.ds(i * 16, 16)
            idxv = sp[sl]
            dd[sl] = sd[sl] + plsc.load_gather(sd, [idxv])
            dp[sl] = plsc.load_gather(sp, [idxv])
            return 0

        lax.fori_loop(0, NPAD // 16, _round, 0)
    sl_out = pl.ds(wid * (NPAD // 32), NPAD // 32)
    pltpu.sync_copy(da.at[sl_out], d_out.at[sl_out])


def _depth_sc(par_pad):
    mesh = plsc.VectorSubcoreMesh(core_axis_name="c", subcore_axis_name="s")
    fn = pl.kernel(
        _depth_body,
        mesh=mesh,
        out_type=jax.ShapeDtypeStruct((NPAD,), jnp.int32),
        scratch_types=[pltpu.VMEM((NPAD,), jnp.int32) for _ in range(4)],
        compiler_params=pltpu.CompilerParams(needs_layout_passes=False),
    )
    return fn(par_pad)


def _search_body(keys_hbm, q_hbm, out_hbm, kv, qv, ov):
    cid = lax.axis_index("c")
    sid = lax.axis_index("s")
    wid = sid * 2 + cid
    per_w = NQPAD // 32
    base = wid * per_w
    pltpu.sync_copy(keys_hbm, kv)
    pltpu.sync_copy(q_hbm.at[pl.ds(base, per_w)], qv)

    def _srch(i, _):
        sl = pl.ds(i * 16, 16)
        q = qv[sl]
        pos = jnp.zeros((16,), jnp.int32)
        for step in (8192, 4096, 2048, 1024, 512, 256, 128, 64,
                     32, 16, 8, 4, 2, 1):
            npos = pos + step
            ok = npos <= N_NODES
            idx = jnp.minimum(npos, N_NODES) - 1
            kvals = plsc.load_gather(kv, [idx])
            pos = jnp.where(ok & (kvals < q), npos, pos)
        ov[sl] = pos
        return 0

    lax.fori_loop(0, per_w // 16, _srch, 0)
    pltpu.sync_copy(ov, out_hbm.at[pl.ds(base, per_w)])


def _search_sc(keys_sorted, queries):
    mesh = plsc.VectorSubcoreMesh(core_axis_name="c", subcore_axis_name="s")
    fn = pl.kernel(
        _search_body,
        mesh=mesh,
        out_type=jax.ShapeDtypeStruct((NQPAD,), jnp.int32),
        scratch_types=[
            pltpu.VMEM((N_NODES,), jnp.int32),
            pltpu.VMEM((NQPAD // 32,), jnp.int32),
            pltpu.VMEM((NQPAD // 32,), jnp.int32),
        ],
        compiler_params=pltpu.CompilerParams(needs_layout_passes=False),
    )
    return fn(keys_sorted, queries)


def _proj_body(x_ref, w_ref, b_ref, oiou_ref, of_ref):
    p = (jnp.dot(x_ref[...], w_ref[...], preferred_element_type=jnp.float32)
         + b_ref[...])
    oiou_ref[...] = p[:, 0:384]
    of_ref[...] = p[:, 384:512]


def _tree_body(xiou_ref, xf_ref, uiou_ref, uf_ref,
               ndp_ref, rng_ref, lvl_ref, prm_ref,
               h_out, c_out, pref, hc_s, st_s, xi_s, xp_s):
    n_levels = prm_ref[0]

    pref[0:8, :] = jnp.zeros((8, 2 * H_DIM), dtype=jnp.float32)

    def _level(t, carry):
        s = lvl_ref[t]
        e = lvl_ref[t + 1]
        last = e - 1
        nch = (e - s + (CHUNK - 1)) // CHUNK

        def _chunk(ci, cin):
            base = s + ci * CHUNK
            nds = []
            for r in range(CHUNK):
                gpos = base + r
                safe = jnp.minimum(gpos, last)
                ndp = ndp_ref[safe]
                rng = rng_ref[safe]
                nd = lax.shift_right_logical(ndp, 14)
                p = lax.bitwise_and(ndp, 16383)
                a0 = lax.shift_right_logical(rng, 14)
                a1 = lax.bitwise_and(rng, 16383)
                nds.append(nd)
                st_s[r:r + 1, :] = (pref[pl.ds(a1, 1), :]
                                    - pref[pl.ds(a0, 1), :])
                xi_s[r:r + 1, :] = xiou_ref[pl.ds(nd, 1), :]
                xp_s[r:r + 1, :] = xf_ref[pl.ds(p, 1), :]

            st = st_s[...]
            iou = xi_s[...] + jnp.dot(st[:, 0:128], uiou_ref[...],
                                      preferred_element_type=jnp.float32)
            c = (jax.nn.sigmoid(iou[:, 0:128]) * jnp.tanh(iou[:, 256:384])
                 + st[:, 128:256])
            h = jax.nn.sigmoid(iou[:, 128:256]) * jnp.tanh(c)
            uf = jnp.dot(h, uf_ref[...], preferred_element_type=jnp.float32)
            f = jax.nn.sigmoid(xp_s[...] + uf)
            hc = jnp.concatenate([h, c], axis=1)
            contrib = jnp.concatenate([h, f * c], axis=1)

            rows = jax.lax.broadcasted_iota(jnp.int32, (CHUNK, 1), 0)
            contrib = jnp.where(rows < (e - base), contrib, 0.0)

            z = jnp.zeros_like(contrib)
            cum = contrib
            sh = 1
            while sh < CHUNK:
                cum = cum + jnp.concatenate([z[0:sh], cum[0:CHUNK - sh]],
                                            axis=0)
                sh *= 2
            cum = cum + cin

            for r in range(CHUNK):
                pref[pl.ds(base + r + 1, 1), :] = cum[r:r + 1, :]
                hc_s[pl.ds(nds[r], 1), :] = hc[r:r + 1, :]
            return cum[CHUNK - 1:CHUNK, :]

        return jax.lax.fori_loop(0, nch, _chunk, carry)

    jax.lax.fori_loop(0, n_levels, _level,
                      jnp.zeros((1, 2 * H_DIM), jnp.float32))

    rows = 1000 if N_NODES % 1000 == 0 else N_NODES

    def _split(i, _):
        blk = hc_s[pl.ds(i * rows, rows), :]
        h_out[pl.ds(i * rows, rows), :] = blk[:, 0:128]
        c_out[pl.ds(i * rows, rows), :] = blk[:, 128:256]
        return 0

    jax.lax.fori_loop(0, N_NODES // rows, _split, 0, unroll=True)


def kernel(input_embeddings, parent, root, W_iou, b_iou, U_iou, W_f, b_f, U_f):
    del root
    x = input_embeddings.astype(jnp.float32)
    n = N_NODES
    idx = jnp.arange(n, dtype=jnp.int32)
    par = parent.astype(jnp.int32)

    d = _depth_sc(jnp.zeros((NPAD,), jnp.int32).at[:n].set(par))[:n]
    maxd = jnp.max(d)
    key = (maxd - d) * n + par
    keys_s, order = lax.sort((key, idx), num_keys=1)
    level_s = keys_s // n
    spar = jnp.where(order == 0, N_NODES, keys_s - level_s * n)
    qk_cs = (level_s - 1) * n + order
    qk_lvl = jnp.arange(n + 16, dtype=jnp.int32) * n
    queries = jnp.concatenate([
        qk_cs, qk_cs + 1, qk_lvl,
        jnp.zeros((NQPAD - 3 * n - 16,), jnp.int32),
    ])
    res = _search_sc(keys_s, queries)
    empty = res[:n] == res[n:2 * n]
    cs_s = jnp.where(empty, 0, res[:n])
    ce_s = jnp.where(empty, 0, res[n:2 * n])
    lvl = res[2 * n:3 * n + 16]
    prm = (maxd + 1).reshape(1)
    ndp = order * 16384 + spar
    rng = cs_s * 16384 + ce_s

    wcat = jnp.concatenate([W_iou.T, W_f.T], axis=1).astype(jnp.float32)
    bcat = jnp.concatenate([b_iou, b_f]).reshape(1, 4 * H_DIM)
    xpad = jnp.zeros((NPAD, D_IN), jnp.float32).at[:n].set(x)
    xiou, xf = pl.pallas_call(
        _proj_body,
        grid=(NPAD // 1024,),
        in_specs=[
            pl.BlockSpec((1024, D_IN), lambda i: (i, 0)),
            pl.BlockSpec((D_IN, 4 * H_DIM), lambda i: (0, 0)),
            pl.BlockSpec((1, 4 * H_DIM), lambda i: (0, 0)),
        ],
        out_specs=(
            pl.BlockSpec((1024, 3 * H_DIM), lambda i: (i, 0)),
            pl.BlockSpec((1024, H_DIM), lambda i: (i, 0)),
        ),
        out_shape=(
            jax.ShapeDtypeStruct((NPAD, 3 * H_DIM), jnp.float32),
            jax.ShapeDtypeStruct((NPAD, H_DIM), jnp.float32),
        ),
    )(xpad, wcat, bcat)

    h, c = pl.pallas_call(
        _tree_body,
        in_specs=(
            [pl.BlockSpec(memory_space=pltpu.VMEM) for _ in range(4)]
            + [pl.BlockSpec(memory_space=pltpu.SMEM) for _ in range(4)]
        ),
        out_specs=(
            pl.BlockSpec(memory_space=pltpu.VMEM),
            pl.BlockSpec(memory_space=pltpu.VMEM),
        ),
        out_shape=(
            jax.ShapeDtypeStruct((N_NODES, H_DIM), jnp.float32),
            jax.ShapeDtypeStruct((N_NODES, H_DIM), jnp.float32),
        ),
        scratch_shapes=[
            pltpu.VMEM((NPAD, 2 * H_DIM), jnp.float32),
            pltpu.VMEM((NPAD, 2 * H_DIM), jnp.float32),
            pltpu.VMEM((CHUNK, 2 * H_DIM), jnp.float32),
            pltpu.VMEM((CHUNK, 3 * H_DIM), jnp.float32),
            pltpu.VMEM((CHUNK, H_DIM), jnp.float32),
        ],
        compiler_params=pltpu.CompilerParams(
            vmem_limit_bytes=100 * 1024 * 1024,
        ),
    )(xiou, xf, U_iou.T.astype(jnp.float32), U_f.T.astype(jnp.float32),
      ndp, rng, lvl, prm)
    return h, c


# speedup vs baseline: 444.8980x; 1.3688x over previous
import jax
import jax.numpy as jnp
from jax import lax
from jax.experimental import pallas as pl
from jax.experimental.pallas import tpu as pltpu
from jax.experimental.pallas import tpu_sc as plsc

N_NODES = 10000
D_IN = 128
H_DIM = 128
NPAD = 10240
NQPAD = 30720
CHUNK = 16


def _depth_body(par_hbm, key_out, pa, pb, da, db, pr, smx):
    cid = lax.axis_index("c")
    sid = lax.axis_index("s")
    wid = sid * 2 + cid
    pltpu.sync_copy(par_hbm, pa)
    pltpu.sync_copy(par_hbm, pr)

    def _init(i, _):
        da[pl.ds(i * 16, 16)] = jnp.ones((16,), jnp.int32)
        return 0

    lax.fori_loop(0, NPAD // 16, _init, 0)
    da[pl.ds(0, 16)] = jnp.minimum(lax.iota(jnp.int32, 16), 1)
    smx[0] = 1

    for kp in range(7):

        @pl.when(smx[0] != 0)
        def _pair():
            def _round_ab(i, _):
                sl = pl.ds(i * 16, 16)
                idxv = pa[sl]
                db[sl] = da[sl] + plsc.load_gather(da, [idxv])
                pb[sl] = plsc.load_gather(pa, [idxv])
                return 0

            lax.fori_loop(0, NPAD // 16, _round_ab, 0)

            def _round_ba(i, m):
                sl = pl.ds(i * 16, 16)
                idxv = pb[sl]
                da[sl] = db[sl] + plsc.load_gather(db, [idxv])
                pv = plsc.load_gather(pb, [idxv])
                pa[sl] = pv
                return jnp.maximum(m, pv)

            mx = lax.fori_loop(0, NPAD // 16, _round_ba,
                               jnp.zeros((16,), jnp.int32))
            smx[0] = jnp.max(mx)

    def _mx(i, m):
        return jnp.maximum(m, da[pl.ds(i * 16, 16)])

    mx = lax.fori_loop(0, NPAD // 16, _mx, jnp.zeros((16,), jnp.int32))
    maxd = jnp.max(mx)

    def _key(i, _):
        sl = pl.ds(i * 16, 16)
        da[sl] = (maxd - da[sl]) * N_NODES + pr[sl]
        return 0

    lax.fori_loop(0, NPAD // 16, _key, 0)
    sl_out = pl.ds(wid * (NPAD // 32), NPAD // 32)
    pltpu.sync_copy(da.at[sl_out], key_out.at[sl_out])


def _depth_sc(par_pad):
    mesh = plsc.VectorSubcoreMesh(core_axis_name="c", subcore_axis_name="s")
    fn = pl.kernel(
        _depth_body,
        mesh=mesh,
        out_type=jax.ShapeDtypeStruct((NPAD,), jnp.int32),
        scratch_types=(
            [pltpu.VMEM((NPAD,), jnp.int32) for _ in range(5)]
            + [pltpu.SMEM((1,), jnp.int32)]
        ),
        compiler_params=pltpu.CompilerParams(needs_layout_passes=False),
    )
    return fn(par_pad)


def _lower_bound(kv, q):
    pos = jnp.zeros((16,), jnp.int32)
    for step in (8192, 4096, 2048, 1024, 512, 256, 128, 64,
                 32, 16, 8, 4, 2, 1):
        npos = pos + step
        ok = npos <= NPAD
        idx = jnp.minimum(npos, NPAD) - 1
        kvals = plsc.load_gather(kv, [idx])
        pos = jnp.where(ok & (kvals < q), npos, pos)
    return pos


def _search_body(keys_hbm, ord_hbm, ndp_out, rng_out, lvl_out,
                 kv, odv, t1, t2, t3):
    cid = lax.axis_index("c")
    sid = lax.axis_index("s")
    wid = sid * 2 + cid
    per_w = NPAD // 32
    base = wid * per_w
    pltpu.sync_copy(keys_hbm, kv)
    pltpu.sync_copy(ord_hbm.at[pl.ds(base, per_w)], odv)

    def _sched(i, _):
        sl = pl.ds(i * 16, 16)
        ks = kv[pl.ds(base + i * 16, 16)]
        od = odv[sl]
        level = ks // N_NODES
        spar = jnp.where(od == 0, N_NODES, ks - level * N_NODES)
        qk = (level - 1) * N_NODES + od
        cs = _lower_bound(kv, qk)
        ce = _lower_bound(kv, qk + 1)
        leaf = cs == ce
        cs = jnp.where(leaf, 0, cs)
        ce = jnp.where(leaf, 0, ce)
        t1[sl] = od * 16384 + spar
        t2[sl] = cs * 16384 + ce
        t3[sl] = _lower_bound(kv, (base + i * 16
                                   + lax.iota(jnp.int32, 16)) * N_NODES)
        return 0

    lax.fori_loop(0, per_w // 16, _sched, 0)
    osl = pl.ds(base, per_w)
    pltpu.sync_copy(t1, ndp_out.at[osl])
    pltpu.sync_copy(t2, rng_out.at[osl])
    pltpu.sync_copy(t3, lvl_out.at[osl])


def _search_sc(keys_pad, order_pad):
    mesh = plsc.VectorSubcoreMesh(core_axis_name="c", subcore_axis_name="s")
    fn = pl.kernel(
        _search_body,
        mesh=mesh,
        out_type=(
            jax.ShapeDtypeStruct((NPAD,), jnp.int32),
            jax.ShapeDtypeStruct((NPAD,), jnp.int32),
            jax.ShapeDtypeStruct((NPAD,), jnp.int32),
        ),
        scratch_types=(
            [pltpu.VMEM((NPAD,), jnp.int32)]
            + [pltpu.VMEM((NPAD // 32,), jnp.int32) for _ in range(4)]
        ),
        compiler_params=pltpu.CompilerParams(needs_layout_passes=False),
    )
    return fn(keys_pad, order_pad)


def _proj_body(x_ref, w_ref, b_ref, oiou_ref, of_ref):
    p = (jnp.dot(x_ref[...], w_ref[...], preferred_element_type=jnp.float32)
         + b_ref[...])
    oiou_ref[...] = p[:, 0:384]
    of_ref[...] = p[:, 384:512]


def _tree_body(xiou_ref, xf_ref, uiou_ref, uf_ref,
               ndp_ref, rng_ref, lvl_ref, prm_ref,
               h_out, c_out, pref, hc_s,
               st_a, xi_a, xp_a, st_b, xi_b, xp_b):
    n_levels = prm_ref[0]

    pref[0:8, :] = jnp.zeros((8, 2 * H_DIM), dtype=jnp.float32)

    def _level(t, carry):
        s = lvl_ref[t]
        e = lvl_ref[t + 1]
        last = e - 1
        npair = (e - s + (2 * CHUNK - 1)) // (2 * CHUNK)

        def _gather(base, st_s, xi_s, xp_s):
            nds = []
            for r in range(CHUNK):
                safe = jnp.minimum(base + r, last)
                ndp = ndp_ref[safe]
                rng = rng_ref[safe]
                nd = lax.shift_right_logical(ndp, 14)
                p = lax.bitwise_and(ndp, 16383)
                a0 = lax.shift_right_logical(rng, 14)
                a1 = lax.bitwise_and(rng, 16383)
                nds.append(nd)
                st_s[r:r + 1, :] = (pref[pl.ds(a1, 1), :]
                                    - pref[pl.ds(a0, 1), :])
                xi_s[r:r + 1, :] = xiou_ref[pl.ds(nd, 1), :]
                xp_s[r:r + 1, :] = xf_ref[pl.ds(p, 1), :]
            return nds

        def _compute(base, cin, st_s, xi_s, xp_s):
            st = st_s[...]
            iou = xi_s[...] + jnp.dot(st[:, 0:128], uiou_ref[...],
                                      preferred_element_type=jnp.float32)
            c = (jax.nn.sigmoid(iou[:, 0:128]) * jnp.tanh(iou[:, 256:384])
                 + st[:, 128:256])
            h = jax.nn.sigmoid(iou[:, 128:256]) * jnp.tanh(c)
            uf = jnp.dot(h, uf_ref[...], preferred_element_type=jnp.float32)
            f = jax.nn.sigmoid(xp_s[...] + uf)
            hc = jnp.concatenate([h, c], axis=1)
            contrib = jnp.concatenate([h, f * c], axis=1)
            rows = jax.lax.broadcasted_iota(jnp.int32, (CHUNK, 1), 0)
            contrib = jnp.where(rows < (e - base), contrib, 0.0)
            z = jnp.zeros_like(contrib)
            cum = contrib
            sh = 1
            while sh < CHUNK:
                cum = cum + jnp.concatenate([z[0:sh], cum[0:CHUNK - sh]],
                                            axis=0)
                sh *= 2
            return hc, cum + cin

        def _scatter(base, nds, hc, cum):
            for r in range(CHUNK):
                pref[pl.ds(base + r + 1, 1), :] = cum[r:r + 1, :]
                hc_s[pl.ds(nds[r], 1), :] = hc[r:r + 1, :]

        def _pair(ci, cin):
            base_a = s + ci * (2 * CHUNK)
            base_b = base_a + CHUNK
            nds_a = _gather(base_a, st_a, xi_a, xp_a)
            nds_b = _gather(base_b, st_b, xi_b, xp_b)
            hc_a, cum_a = _compute(base_a, cin, st_a, xi_a, xp_a)
            hc_b, cum_b = _compute(base_b, cum_a[CHUNK - 1:CHUNK, :],
                                   st_b, xi_b, xp_b)
            _scatter(base_a, nds_a, hc_a, cum_a)
            _scatter(base_b, nds_b, hc_b, cum_b)
            return cum_b[CHUNK - 1:CHUNK, :]

        return jax.lax.fori_loop(0, npair, _pair, carry)

    jax.lax.fori_loop(0, n_levels, _level,
                      jnp.zeros((1, 2 * H_DIM), jnp.float32))

    rows = 1000 if N_NODES % 1000 == 0 else N_NODES

    def _split(i, _):
        blk = hc_s[pl.ds(i * rows, rows), :]
        h_out[pl.ds(i * rows, rows), :] = blk[:, 0:128]
        c_out[pl.ds(i * rows, rows), :] = blk[:, 128:256]
        return 0

    jax.lax.fori_loop(0, N_NODES // rows, _split, 0, unroll=True)


def kernel(input_embeddings, parent, root, W_iou, b_iou, U_iou, W_f, b_f, U_f):
    del root
    x = input_embeddings.astype(jnp.float32)
    n = N_NODES
    idx = jnp.arange(n, dtype=jnp.int32)
    par = parent.astype(jnp.int32)

    key = _depth_sc(jnp.zeros((NPAD,), jnp.int32).at[:n].set(par))[:n]
    keys_s, order = lax.sort((key, idx), num_keys=1)
    big = jnp.int32(1 << 30)
    keys_pad = jnp.concatenate([keys_s, jnp.full((NPAD - n,), big,
                                                 jnp.int32)])
    order_pad = jnp.concatenate([order, jnp.zeros((NPAD - n,), jnp.int32)])
    ndp, rng, lvl_full = _search_sc(keys_pad, order_pad)
    lvl = lvl_full[:n + 16]
    prm = (keys_s[n - 1] // n + 1).reshape(1)

    wcat = jnp.concatenate([W_iou.T, W_f.T], axis=1).astype(jnp.float32)
    bcat = jnp.concatenate([b_iou, b_f]).reshape(1, 4 * H_DIM)
    xpad = jnp.zeros((NPAD, D_IN), jnp.float32).at[:n].set(x)
    xiou, xf = pl.pallas_call(
        _proj_body,
        grid=(NPAD // 1024,),
        in_specs=[
            pl.BlockSpec((1024, D_IN), lambda i: (i, 0)),
            pl.BlockSpec((D_IN, 4 * H_DIM), lambda i: (0, 0)),
            pl.BlockSpec((1, 4 * H_DIM), lambda i: (0, 0)),
        ],
        out_specs=(
            pl.BlockSpec((1024, 3 * H_DIM), lambda i: (i, 0)),
            pl.BlockSpec((1024, H_DIM), lambda i: (i, 0)),
        ),
        out_shape=(
            jax.ShapeDtypeStruct((NPAD, 3 * H_DIM), jnp.float32),
            jax.ShapeDtypeStruct((NPAD, H_DIM), jnp.float32),
        ),
    )(xpad, wcat, bcat)

    h, c = pl.pallas_call(
        _tree_body,
        in_specs=(
            [pl.BlockSpec(memory_space=pltpu.VMEM) for _ in range(4)]
            + [pl.BlockSpec(memory_space=pltpu.SMEM) for _ in range(4)]
        ),
        out_specs=(
            pl.BlockSpec(memory_space=pltpu.VMEM),
            pl.BlockSpec(memory_space=pltpu.VMEM),
        ),
        out_shape=(
            jax.ShapeDtypeStruct((N_NODES, H_DIM), jnp.float32),
            jax.ShapeDtypeStruct((N_NODES, H_DIM), jnp.float32),
        ),
        scratch_shapes=[
            pltpu.VMEM((NPAD, 2 * H_DIM), jnp.float32),
            pltpu.VMEM((NPAD, 2 * H_DIM), jnp.float32),
            pltpu.VMEM((CHUNK, 2 * H_DIM), jnp.float32),
            pltpu.VMEM((CHUNK, 3 * H_DIM), jnp.float32),
            pltpu.VMEM((CHUNK, H_DIM), jnp.float32),
            pltpu.VMEM((CHUNK, 2 * H_DIM), jnp.float32),
            pltpu.VMEM((CHUNK, 3 * H_DIM), jnp.float32),
            pltpu.VMEM((CHUNK, H_DIM), jnp.float32),
        ],
        compiler_params=pltpu.CompilerParams(
            vmem_limit_bytes=100 * 1024 * 1024,
        ),
    )(xiou, xf, U_iou.T.astype(jnp.float32), U_f.T.astype(jnp.float32),
      ndp, rng, lvl, prm)
    return h, c
